# baseline (speedup 1.0000x reference)
import functools

import jax
import jax.numpy as jnp
from jax import lax
from jax.experimental import pallas as pl
from jax.experimental.pallas import tpu as pltpu
from jax.experimental.pallas import tpu_sc as plsc

_B = 128
_V = 100000
_L = 16
_NC = 2
_NS = 16
_RG = 8
_CW = 3200
_NCH = 15
_TAIL0 = (96000, 2048, 128)
_TAIL1 = (98048, 1920, 120)
_VT = _V - 32

_mesh = plsc.VectorSubcoreMesh(core_axis_name="c", subcore_axis_name="s")


@functools.partial(
    pl.kernel,
    mesh=_mesh,
    out_type=[
        jax.ShapeDtypeStruct((_B,), jnp.int32),
        jax.ShapeDtypeStruct((_B, _V), jnp.float32),
    ],
    scratch_types=[
        pltpu.VMEM((_RG, _CW), jnp.float32),
        pltpu.VMEM((_RG, _CW), jnp.float32),
        pltpu.VMEM((_RG, _CW), jnp.float32),
        pltpu.VMEM((_RG, 128), jnp.float32),
        pltpu.VMEM((_L,), jnp.float32),
        pltpu.VMEM((_L,), jnp.int32),
        pltpu.VMEM((_L,), jnp.float32),
        pltpu.VMEM((_L,), jnp.int32),
        pltpu.VMEM((_L,), jnp.int32),
        pltpu.VMEM((_RG * _L,), jnp.float32),
        pltpu.VMEM((_RG * _L,), jnp.int32),
        pltpu.VMEM_SHARED((_NS * _L,), jnp.float32),
        pltpu.VMEM_SHARED((_NS * _L,), jnp.int32),
        pltpu.SemaphoreType.DMA,
        pltpu.SemaphoreType.DMA,
        pltpu.SemaphoreType.DMA,
    ],
)
def _argmax_onehot_sc(in_hbm, tail_hbm, idx_hbm, enc_hbm, buf0, buf1, zbuf,
                      tbuf, fvbuf, ivbuf, nfbuf, nibuf, mbuf, fabuf, iabuf,
                      sharedf, sharedi, sem0, sem1, semz):
    c = lax.axis_index("c")
    s = lax.axis_index("s")
    half = s % 2
    row0 = pl.multiple_of((c * (_NS // 2) + s // 2) * _RG, _RG)
    lanes = lax.broadcasted_iota(jnp.int32, (_L,), 0)
    zvec = jnp.zeros((_L,), jnp.float32)
    one = jnp.float32(1.0)

    def zero_body(j, carry):
        for rr in range(_RG):
            zbuf[rr, pl.ds(j * _L, _L)] = zvec
        return carry

    lax.fori_loop(0, _CW // _L, zero_body, 0)

    def tzero_body(j, carry):
        for rr in range(_RG):
            tbuf[rr, pl.ds(j * _L, _L)] = zvec
        return carry

    lax.fori_loop(0, 128 // _L, tzero_body, 0)

    def chunk_off(k):
        return pl.multiple_of(k * (2 * _CW) + half * _CW, 128)

    zcopies = [
        pltpu.async_copy(
            zbuf, enc_hbm.at[pl.ds(row0, _RG), pl.ds(chunk_off(k), _CW)],
            semz)
        for k in range(_NCH)
    ]
    tail_zc = {}
    for h, (toff, tw, _nv) in ((0, _TAIL0), (1, _TAIL1)):
        @pl.when(half == h)
        def _(toff=toff, tw=tw, h=h):
            tail_zc[h] = pltpu.async_copy(
                zbuf.at[:, pl.ds(0, tw)],
                enc_hbm.at[pl.ds(row0, _RG), pl.ds(toff, tw)], semz)

    bufs = (buf0, buf1)
    sems = (sem0, sem1)

    def start(k):
        return pltpu.async_copy(
            in_hbm.at[pl.ds(row0, _RG), pl.ds(chunk_off(k), _CW)],
            bufs[k % 2], sems[k % 2])

    def make_scan(buf, nvregs, g0):
        def body(j, carry):
            bests, bvrs = list(carry[0]), list(carry[1])
            gv = jnp.full((_L,), g0 + j, jnp.int32)
            for rr in range(_RG):
                v = buf[rr, pl.ds(j * _L, _L)]
                m = v > bests[rr]
                bests[rr] = jnp.where(m, v, bests[rr])
                bvrs[rr] = jnp.where(m, gv, bvrs[rr])
            return tuple(bests), tuple(bvrs)

        return lambda accs: lax.fori_loop(0, nvregs, body, accs)

    neg_inf = jnp.full((_L,), -jnp.inf, jnp.float32)
    izero = jnp.zeros((_L,), jnp.int32)
    accs = ((neg_inf,) * _RG, (izero,) * _RG)

    pending = start(0)
    for k in range(_NCH):
        pending.wait()
        nxt = start(k + 1) if k + 1 < _NCH else None
        g0 = k * (2 * _CW // _L) + half * (_CW // _L)
        accs = make_scan(bufs[k % 2], _CW // _L, g0)(accs)
        pending = nxt

    def acc_store(a):
        bs, vs = a
        for rr in range(_RG):
            fabuf[pl.ds(rr * _L, _L)] = bs[rr]
            iabuf[pl.ds(rr * _L, _L)] = vs[rr]

    def acc_load():
        return (tuple(fabuf[pl.ds(rr * _L, _L)] for rr in range(_RG)),
                tuple(iabuf[pl.ds(rr * _L, _L)] for rr in range(_RG)))

    acc_store(accs)

    @pl.when(half == 0)
    def _():
        toff, tw, nv = _TAIL0
        pltpu.sync_copy(
            in_hbm.at[pl.ds(row0, _RG), pl.ds(toff, tw)],
            buf0.at[:, pl.ds(0, tw)])
        acc_store(make_scan(buf0, nv, toff // _L)(acc_load()))

    @pl.when(half == 1)
    def _():
        toff, tw, nv = _TAIL1
        pltpu.sync_copy(
            in_hbm.at[pl.ds(row0, _RG), pl.ds(toff, tw)],
            buf0.at[:, pl.ds(0, tw)])
        a = make_scan(buf0, nv, toff // _L)(acc_load())
        pltpu.sync_copy(tail_hbm.at[pl.ds(row0, _RG)],
                        buf1.at[:, pl.ds(0, 128)])
        acc_store(make_scan(buf1, 2, _VT // _L)(a))

    bests, bvrs = acc_load()

    def tree_max(v):
        for st in (8, 4, 2, 1):
            v = jnp.maximum(v, jnp.take(v, (lanes + st) & 15))
        return v

    fvec = neg_inf
    ivec = izero
    big = jnp.full((_L,), _V, jnp.int32)
    for rr in range(_RG):
        rmax = tree_max(bests[rr])
        cand = jnp.where(bests[rr] == rmax, bvrs[rr] * _L + lanes, big)
        ridx = -tree_max(-cand)
        fvec = jnp.where(lanes == rr, rmax, fvec)
        ivec = jnp.where(lanes == rr, ridx, ivec)

    fvbuf[...] = fvec
    ivbuf[...] = ivec
    pltpu.sync_copy(fvbuf, sharedf.at[pl.ds(s * _L, _L)])
    pltpu.sync_copy(ivbuf, sharedi.at[pl.ds(s * _L, _L)])

    for zc in zcopies:
        zc.wait()
    for h in (0, 1):
        @pl.when(half == h)
        def _(h=h):
            tail_zc[h].wait()
    plsc.subcore_barrier()

    @pl.when(half == 0)
    def _():
        pltpu.sync_copy(sharedf.at[pl.ds((s + 1) * _L, _L)], nfbuf)
        pltpu.sync_copy(sharedi.at[pl.ds((s + 1) * _L, _L)], nibuf)
        nf = nfbuf[...]
        ni = nibuf[...]
        sel = (nf > fvec) | ((nf == fvec) & (ni < ivec))
        mi = jnp.where(sel, ni, ivec)
        mbuf[...] = mi
        pltpu.sync_copy(mbuf.at[pl.ds(0, _RG)], idx_hbm.at[pl.ds(row0, _RG)])

        last_tile = _VT // 128 - 1
        tcv = mi >> 7
        citv = mi & 127
        ltv = jnp.full((_L,), last_tile, jnp.int32)
        tcwv = jnp.where(tcv > ltv, ltv, tcv)
        valid = lanes < _RG

        for r in range(_RG):
            rb = jnp.full((_L,), r, jnp.int32)
            tcwb = jnp.take(tcwv, rb)
            matchm = jnp.where((tcv == tcwb) & valid, 1, 0)
            qoffs = []
            for rp in range(_RG):
                rpb = jnp.full((_L,), rp, jnp.int32)
                mb = jnp.take(matchm, rpb)
                citb = jnp.take(citv, rpb)
                qoff = (citv[rp] // _L) * _L
                qoffs.append(qoff)
                e = jnp.where(lanes + qoff == citb + (1 - mb) * 256,
                              one, zvec)
                tbuf[rp, pl.ds(qoff, _L)] = e
            c0 = (tcwv * 128)[r]
            pltpu.sync_copy(
                tbuf,
                enc_hbm.at[pl.ds(row0, _RG),
                           pl.ds(pl.multiple_of(c0, 128), 128)])
            for rp in range(_RG):
                tbuf[rp, pl.ds(qoffs[rp], _L)] = zvec


def kernel(inputs):
    tail = jnp.pad(lax.slice(inputs, (0, _VT), (_B, _V)), ((0, 0), (0, 96)))
    indices, enc = _argmax_onehot_sc(inputs, tail)
    cols = lax.broadcasted_iota(jnp.int32, (_B, _V - _VT), 1) + _VT
    tail_onehot = (cols == indices[:, None]).astype(jnp.float32)
    enc = enc
    return (indices, enc)


# speedup vs baseline: 1.3881x; 1.3881x over previous
import functools

import jax
import jax.numpy as jnp
from jax import lax
from jax.experimental import pallas as pl
from jax.experimental.pallas import tpu as pltpu
from jax.experimental.pallas import tpu_sc as plsc

_B = 128
_V = 100000
_N = _B * _V
_L = 16
_NC = 2
_NS = 16
_NW = _NC * _NS
_WPW = _N // _NW
_CH = 16000
_NCHK = _WPW // _CH
_VRC = _CH // _L
_NSTR = _B // _L
_NP = _NSTR * _NW * _L

_mesh = plsc.VectorSubcoreMesh(core_axis_name="c", subcore_axis_name="s")


@functools.partial(
    pl.kernel,
    mesh=_mesh,
    out_type=[
        jax.ShapeDtypeStruct((_NP,), jnp.float32),
        jax.ShapeDtypeStruct((_NP,), jnp.int32),
        jax.ShapeDtypeStruct((_N,), jnp.float32),
    ],
    scratch_types=[
        pltpu.VMEM((_CH,), jnp.float32),
        pltpu.VMEM((_CH,), jnp.float32),
        pltpu.VMEM((_CH,), jnp.float32),
        pltpu.VMEM((_B,), jnp.float32),
        pltpu.VMEM((_B,), jnp.int32),
        pltpu.SemaphoreType.DMA,
        pltpu.SemaphoreType.DMA,
        pltpu.SemaphoreType.DMA,
    ],
)
def _scan_zerofill_sc(in_hbm, pf_hbm, pi_hbm, enc_hbm, buf0, buf1, zbuf,
                      stf, sti, sem0, sem1, semz):
    w = lax.axis_index("c") * _NS + lax.axis_index("s")
    base = w * _WPW
    zvec = jnp.zeros((_L,), jnp.float32)

    def zero_body(j, carry):
        zbuf[pl.ds(j * _L, _L)] = zvec
        return carry

    lax.fori_loop(0, _VRC, zero_body, 0)

    zcopies = [
        pltpu.async_copy(
            zbuf, enc_hbm.at[pl.ds(base + k * _CH, _CH)], semz)
        for k in range(_NCHK)
    ]

    bufs = (buf0, buf1)
    sems = (sem0, sem1)

    def start(k):
        return pltpu.async_copy(
            in_hbm.at[pl.ds(base + k * _CH, _CH)], bufs[k % 2], sems[k % 2])

    neg_inf = jnp.full((_L,), -jnp.inf, jnp.float32)
    izero = jnp.zeros((_L,), jnp.int32)
    best = [neg_inf] * _NSTR
    bg = [izero] * _NSTR

    pending = start(0)
    for k in range(_NCHK):
        pending.wait()
        nxt = start(k + 1) if k + 1 < _NCHK else None
        buf = bufs[k % 2]
        gbase = w * (_WPW // _L) + k * _VRC

        def body(jj, carry, buf=buf, gbase=gbase):
            bs, gs = list(carry[0]), list(carry[1])
            for u in range(_NSTR):
                j = jj * _NSTR + u
                v = buf[pl.ds(j * _L, _L)]
                m = v > bs[u]
                gv = jnp.full((_L,), gbase + j, jnp.int32)
                bs[u] = jnp.where(m, v, bs[u])
                gs[u] = jnp.where(m, gv, gs[u])
            return tuple(bs), tuple(gs)

        bt, gt = lax.fori_loop(0, _VRC // _NSTR, body,
                               (tuple(best), tuple(bg)))
        best, bg = list(bt), list(gt)
        pending = nxt

    for t in range(_NSTR):
        stf[pl.ds(t * _L, _L)] = best[t]
        sti[pl.ds(t * _L, _L)] = bg[t]
    pcopies = []
    for t in range(_NSTR):
        off = (t * _NW + w) * _L
        pcopies.append(pltpu.async_copy(
            stf.at[pl.ds(t * _L, _L)], pf_hbm.at[pl.ds(off, _L)], semz))
        pcopies.append(pltpu.async_copy(
            sti.at[pl.ds(t * _L, _L)], pi_hbm.at[pl.ds(off, _L)], semz))

    for zc in zcopies:
        zc.wait()
    for pc in pcopies:
        pc.wait()


@functools.partial(
    pl.kernel,
    mesh=_mesh,
    out_type=jax.ShapeDtypeStruct((_B,), jnp.int32),
    scratch_types=[
        pltpu.VMEM((_NW * _L,), jnp.float32),
        pltpu.VMEM((_NW * _L,), jnp.int32),
        pltpu.VMEM((_L,), jnp.int32),
        pltpu.SemaphoreType.DMA,
    ],
)
def _merge_sc(pf_hbm, pi_hbm, idx_hbm, vf, vi, ivbuf, sem):
    w = lax.axis_index("c") * _NS + lax.axis_index("s")

    @pl.when(w < _NSTR)
    def _():
        soff = w * (_NW * _L)
        pltpu.sync_copy(pf_hbm.at[pl.ds(soff, _NW * _L)], vf)
        pltpu.sync_copy(pi_hbm.at[pl.ds(soff, _NW * _L)], vi)
        best = vf[pl.ds(0, _L)]
        bg = vi[pl.ds(0, _L)]
        for m in range(1, _NW):
            ob = vf[pl.ds(m * _L, _L)]
            og = vi[pl.ds(m * _L, _L)]
            sel = (ob > best) | ((ob == best) & (og < bg))
            best = jnp.where(sel, ob, best)
            bg = jnp.where(sel, og, bg)
        ivbuf[...] = bg >> 3
        pltpu.sync_copy(ivbuf, idx_hbm.at[pl.ds(w * _L, _L)])


def kernel(inputs):
    xflat = inputs.T.reshape(_N)
    pf, pi, encz = _scan_zerofill_sc(xflat)
    indices = _merge_sc(pf, pi)
    pos = indices * _B + lax.iota(jnp.int32, _B)
    encflat = encz.at[pos].set(jnp.float32(1.0), mode="promise_in_bounds",
                               unique_indices=True)
    enc = encflat.reshape(_V, _B).T
    return (indices, enc)
